# baseline (speedup 1.0000x reference)
import functools

import jax
import jax.numpy as jnp
from jax import lax
from jax.experimental import pallas as pl
from jax.experimental.pallas import tpu as pltpu
from jax.experimental.pallas import tpu_sc as plsc

D = 128
EPS = 1e-5

NC = 2
NS = 16
NW = NC * NS
CHUNK = 40


def _ln(y, g, b):
    mu = jnp.mean(y, axis=-1, keepdims=True)
    d = y - mu
    var = jnp.mean(d * d, axis=-1, keepdims=True)
    return d * lax.rsqrt(var + EPS) * g + b


def _dot(a, b):
    return jnp.dot(a, b, preferred_element_type=jnp.float32)


def _node_enc_body(x, w1, b1, w2, b2, g, bn, ws, wr, h_o, as_o, ar_o):
    t = jnp.maximum(_dot(x[...], w1[...]) + b1[...], 0.0)
    h = _ln(_dot(t, w2[...]) + b2[...], g[...], bn[...])
    h_o[...] = h
    as_o[...] = _dot(h, ws[...])
    ar_o[...] = _dot(h, wr[...])


def _edge_proj_body(ea, w1, b1, w2, b2, g, bn, we0, c0, we1, c1, ve0_o, ve1_o):
    t = jnp.maximum(_dot(ea[...], w1[...]) + b1[...], 0.0)
    e = _ln(_dot(t, w2[...]) + b2[...], g[...], bn[...])
    ve0_o[...] = _dot(e, we0[...]) + c0[...]
    ve1_o[...] = _dot(e, we1[...]) + c1[...]


def _edge_m_body(m1, w2, b2, g, bn, m_o):
    m_o[...] = _ln(jnp.maximum(_dot(m1[...], w2[...]) + b2[...], 0.0),
                   g[...], bn[...])


def _node_up_body(h, a0, a1, wh, wa, b1, w2, b2, g, bn, ws, wr,
                  h_o, as_o, ar_o):
    agg = a0[...] + a1[...]
    t = jnp.maximum(_dot(h[...], wh[...]) + _dot(agg, wa[...]) + b1[...], 0.0)
    nu = _dot(t, w2[...]) + b2[...]
    hn = _ln(h[...] + nu, g[...], bn[...])
    h_o[...] = hn
    as_o[...] = _dot(hn, ws[...])
    ar_o[...] = _dot(hn, wr[...])


def _node_up_dec_body(h, a0, a1, wh, wa, b1, w2, b2, g, bn, wd1, c1, wd2, c2,
                      out_o):
    agg = a0[...] + a1[...]
    t = jnp.maximum(_dot(h[...], wh[...]) + _dot(agg, wa[...]) + b1[...], 0.0)
    nu = _dot(t, w2[...]) + b2[...]
    hn = _ln(h[...] + nu, g[...], bn[...])
    d = jnp.maximum(_dot(hn, wd1[...]) + c1[...], 0.0)
    out_o[...] = _dot(d, wd2[...]) + c2[...]


def _full(a):
    nd = len(a.shape)
    return pl.BlockSpec(a.shape, lambda i: (0,) * nd)


def _rows(tile, width):
    return pl.BlockSpec((tile, width), lambda i: (i, 0))


def _tc_call(body, row_args, consts, n_rows, tile, out_widths):
    grid = (n_rows // tile,)
    in_specs = [_rows(tile, a.shape[-1]) for a in row_args]
    in_specs += [_full(c) for c in consts]
    out_shape = [jax.ShapeDtypeStruct((n_rows, w), jnp.float32)
                 for w in out_widths]
    out_specs = [_rows(tile, w) for w in out_widths]
    return pl.pallas_call(
        body, grid=grid, in_specs=in_specs, out_specs=out_specs,
        out_shape=out_shape)(*row_args, *consts)


def _sc_gather(as_t, ar_t, ve, s3, r3):
    e = ve.shape[0]
    per_w = e // NW
    n_ch = per_w // CHUNK
    mesh = plsc.VectorSubcoreMesh(core_axis_name="c", subcore_axis_name="s")

    @functools.partial(
        pl.kernel, mesh=mesh,
        out_type=jax.ShapeDtypeStruct((e, D), jnp.float32),
        scratch_types=[
            pltpu.VMEM((n_ch, CHUNK), jnp.int32),
            pltpu.VMEM((n_ch, CHUNK), jnp.int32),
            pltpu.VMEM((2, CHUNK, D), jnp.float32),
            pltpu.VMEM((2, CHUNK, D), jnp.float32),
            pltpu.VMEM((2, CHUNK, D), jnp.float32),
            pltpu.SemaphoreType.DMA,
            pltpu.SemaphoreType.DMA,
            pltpu.SemaphoreType.DMA,
            pltpu.SemaphoreType.DMA,
        ],
    )
    def k(as_h, ar_h, ve_h, si_h, ri_h, out_h, si_v, ri_v, rs_v, rr_v, ve_v,
          g0, g1, o0, o1):
        wid = lax.axis_index("s") * NC + lax.axis_index("c")
        base = wid * per_w
        gsem = (g0, g1)
        osem = (o0, o1)

        pltpu.sync_copy(si_h.at[wid], si_v)
        pltpu.sync_copy(ri_h.at[wid], ri_v)

        def issue_g(b, c):
            pltpu.async_copy(as_h.at[si_v.at[c]], rs_v.at[b], gsem[b])
            pltpu.async_copy(ar_h.at[ri_v.at[c]], rr_v.at[b], gsem[b])
            pltpu.async_copy(ve_h.at[pl.ds(base + c * CHUNK, CHUNK)],
                             ve_v.at[b], gsem[b])

        def wait_g(b):
            for dst in (rs_v, rr_v, ve_v):
                pltpu.make_async_copy(ve_h.at[pl.ds(base, CHUNK)],
                                      dst.at[b], gsem[b]).wait()

        def wait_o(b):
            pltpu.make_async_copy(ve_v.at[b],
                                  out_h.at[pl.ds(base, CHUNK)],
                                  osem[b]).wait()

        issue_g(0, 0)

        def pair(i, carry):
            for b in (0, 1):
                c = 2 * i + b
                other = 1 - b

                @pl.when(c + 1 < n_ch)
                def _prefetch():
                    @pl.when(c >= 1)
                    def _drain():
                        wait_o(other)

                    issue_g(other, c + 1)

                wait_g(b)

                @plsc.parallel_loop(0, CHUNK, unroll=4)
                def _row(j):
                    for kk in range(D // 16):
                        sl = pl.ds(kk * 16, 16)
                        v = (rs_v[b, j, sl] + rr_v[b, j, sl]
                             + ve_v[b, j, sl])
                        ve_v[b, j, sl] = jnp.maximum(v, 0.0)
                pltpu.async_copy(ve_v.at[b],
                                 out_h.at[pl.ds(base + c * CHUNK, CHUNK)],
                                 osem[b])
            return carry

        lax.fori_loop(0, n_ch // 2, pair, 0)
        wait_o(0)
        wait_o(1)

    return k(as_t, ar_t, ve, s3, r3)


def _sc_scatter(m, r3, zeros, n_pad):
    e = m.shape[0]
    per_w = e // NW
    n_ch = per_w // CHUNK
    rows_per_tile = n_pad // NS
    mesh = plsc.VectorSubcoreMesh(core_axis_name="c", subcore_axis_name="s")

    @functools.partial(
        pl.kernel, mesh=mesh,
        out_type=[jax.ShapeDtypeStruct((n_pad, D), jnp.float32),
                  jax.ShapeDtypeStruct((n_pad, D), jnp.float32)],
        scratch_types=[
            pltpu.VMEM_SHARED((n_pad, D), jnp.float32),
            pltpu.VMEM((n_ch, CHUNK), jnp.int32),
            pltpu.VMEM((2, CHUNK, D), jnp.float32),
            pltpu.SemaphoreType.DMA,
            pltpu.SemaphoreType.DMA,
            pltpu.SemaphoreType.DMA,
            pltpu.SemaphoreType.DMA,
        ],
    )
    def k(m_h, ri_h, z_h, out0_h, out1_h, acc_sh, ri_v, m_v, l0, l1, s0, s1):
        cid = lax.axis_index("c")
        sid = lax.axis_index("s")
        wid = sid * NC + cid
        base = wid * per_w
        lsem = (l0, l1)
        ssem = (s0, s1)

        @pl.when(sid == 0)
        def _init():
            pltpu.sync_copy(z_h, acc_sh)

        pltpu.sync_copy(ri_h.at[wid], ri_v)
        plsc.subcore_barrier()

        def issue_m(b, c):
            pltpu.async_copy(m_h.at[pl.ds(base + c * CHUNK, CHUNK)],
                             m_v.at[b], lsem[b])

        def wait_m(b):
            pltpu.make_async_copy(m_h.at[pl.ds(base, CHUNK)],
                                  m_v.at[b], lsem[b]).wait()

        def wait_sc(b):
            pltpu.make_async_copy(m_v.at[b], acc_sh.at[pl.ds(0, CHUNK)],
                                  ssem[b]).wait()

        issue_m(0, 0)

        def pair(i, carry):
            for b in (0, 1):
                c = 2 * i + b
                other = 1 - b

                @pl.when(c + 1 < n_ch)
                def _prefetch():
                    @pl.when(c >= 1)
                    def _drain():
                        wait_sc(other)

                    issue_m(other, c + 1)

                wait_m(b)
                pltpu.async_copy(m_v.at[b], acc_sh.at[ri_v.at[c]], ssem[b],
                                 add=True)
            return carry

        lax.fori_loop(0, n_ch // 2, pair, 0)
        wait_sc(0)
        wait_sc(1)
        plsc.subcore_barrier()

        row0 = sid * rows_per_tile

        @pl.when(cid == 0)
        def _wb0():
            pltpu.sync_copy(acc_sh.at[pl.ds(row0, rows_per_tile)],
                            out0_h.at[pl.ds(row0, rows_per_tile)])

        @pl.when(cid == 1)
        def _wb1():
            pltpu.sync_copy(acc_sh.at[pl.ds(row0, rows_per_tile)],
                            out1_h.at[pl.ds(row0, rows_per_tile)])

    return k(m, r3, zeros)


def kernel(x, edge_index, edge_attr, params):
    n = x.shape[0]
    e = edge_index.shape[1]
    n_pad = ((n + 8 * NS - 1) // (8 * NS)) * (8 * NS)

    per_w = e // NW
    n_ch = per_w // CHUNK
    s3 = edge_index[0].astype(jnp.int32).reshape(NW, n_ch, CHUNK)
    r3 = edge_index[1].astype(jnp.int32).reshape(NW, n_ch, CHUNK)

    def lin(p):
        return p["W"], p["b"].reshape(1, -1)

    ne1w, ne1b = lin(params["node_enc"][0])
    ne2w, ne2b = lin(params["node_enc"][1])
    ng = params["node_enc_ln"]["g"].reshape(1, -1)
    nb = params["node_enc_ln"]["b"].reshape(1, -1)
    ee1w, ee1b = lin(params["edge_enc"][0])
    ee2w, ee2b = lin(params["edge_enc"][1])
    eg = params["edge_enc_ln"]["g"].reshape(1, -1)
    eb = params["edge_enc_ln"]["b"].reshape(1, -1)

    lw = []
    for lp in params["layers"]:
        em1w, em1b = lin(lp["edge_mlp"][0])
        em2w, em2b = lin(lp["edge_mlp"][1])
        nm1w, nm1b = lin(lp["node_mlp"][0])
        nm2w, nm2b = lin(lp["node_mlp"][1])
        lw.append(dict(
            ws=em1w[:D], wr=em1w[D:2 * D], we=em1w[2 * D:], c1=em1b,
            em2w=em2w, em2b=em2b,
            eg=lp["edge_norm"]["g"].reshape(1, -1),
            eb=lp["edge_norm"]["b"].reshape(1, -1),
            wh=nm1w[:D], wa=nm1w[D:], nb1=nm1b,
            nm2w=nm2w, nm2b=nm2b,
            ng=lp["node_norm"]["g"].reshape(1, -1),
            nb2=lp["node_norm"]["b"].reshape(1, -1),
        ))

    d1w, d1b = lin(params["dec"][0])
    d2w, d2b = lin(params["dec"][1])
    d2w = jnp.pad(d2w, ((0, 0), (0, D - d2w.shape[1])))
    d2b = jnp.pad(d2b, ((0, 0), (0, D - d2b.shape[1])))

    tn, te = 1000, 2000

    h, as_t, ar_t = _tc_call(
        _node_enc_body, [x],
        [ne1w, ne1b, ne2w, ne2b, ng, nb, lw[0]["ws"], lw[0]["wr"]],
        n, tn, [D, D, D])

    ve0, ve1 = _tc_call(
        _edge_proj_body, [edge_attr],
        [ee1w, ee1b, ee2w, ee2b, eg, eb,
         lw[0]["we"], lw[0]["c1"], lw[1]["we"], lw[1]["c1"]],
        e, te, [D, D])

    zeros = jnp.zeros((n_pad, D), jnp.float32)
    ve = [ve0, ve1]
    out = None
    for l, w in enumerate(lw):
        m1 = _sc_gather(as_t, ar_t, ve[l], s3, r3)
        m = _tc_call(_edge_m_body, [m1],
                     [w["em2w"], w["em2b"], w["eg"], w["eb"]],
                     e, te, [D])[0]
        a0, a1 = _sc_scatter(m, r3, zeros, n_pad)
        if l == 0:
            nxt = lw[1]
            h, as_t, ar_t = _tc_call(
                _node_up_body, [h, a0[:n], a1[:n]],
                [w["wh"], w["wa"], w["nb1"], w["nm2w"], w["nm2b"],
                 w["ng"], w["nb2"], nxt["ws"], nxt["wr"]],
                n, tn, [D, D, D])
        else:
            out = _tc_call(
                _node_up_dec_body, [h, a0[:n], a1[:n]],
                [w["wh"], w["wa"], w["nb1"], w["nm2w"], w["nm2b"],
                 w["ng"], w["nb2"], d1w, d1b, d2w, d2b],
                n, tn, [D])[0]

    return out[:, :3]


# speedup vs baseline: 1.1800x; 1.1800x over previous
import functools

import jax
import jax.numpy as jnp
from jax import lax
from jax.experimental import pallas as pl
from jax.experimental.pallas import tpu as pltpu
from jax.experimental.pallas import tpu_sc as plsc

D = 128
EPS = 1e-5

NC = 2
NS = 16
NW = NC * NS
CHUNK = 40


def _ln(y, g, b):
    mu = jnp.mean(y, axis=-1, keepdims=True)
    d = y - mu
    var = jnp.mean(d * d, axis=-1, keepdims=True)
    return d * lax.rsqrt(var + EPS) * g + b


def _dot(a, b):
    return jnp.dot(a, b, preferred_element_type=jnp.float32)


def _node_enc_body(x, w1, b1, w2, b2, g, bn, ws, wr, h_o, as_o, ar_o):
    t = jnp.maximum(_dot(x[...], w1[...]) + b1[...], 0.0)
    h = _ln(_dot(t, w2[...]) + b2[...], g[...], bn[...])
    h_o[...] = h
    as_o[...] = _dot(h, ws[...])
    ar_o[...] = _dot(h, wr[...])


def _edge_proj_body(ea, w1, b1, w2, b2, g, bn, we0, c0, we1, c1, ve0_o, ve1_o):
    t = jnp.maximum(_dot(ea[...], w1[...]) + b1[...], 0.0)
    e = _ln(_dot(t, w2[...]) + b2[...], g[...], bn[...])
    ve0_o[...] = _dot(e, we0[...]) + c0[...]
    ve1_o[...] = _dot(e, we1[...]) + c1[...]


def _edge_m_body(m1, w2, b2, g, bn, m_o):
    m_o[...] = _ln(jnp.maximum(_dot(m1[...], w2[...]) + b2[...], 0.0),
                   g[...], bn[...])


def _node_up_body(h, a0, a1, wh, wa, b1, w2, b2, g, bn, ws, wr,
                  h_o, as_o, ar_o):
    agg = a0[...] + a1[...]
    t = jnp.maximum(_dot(h[...], wh[...]) + _dot(agg, wa[...]) + b1[...], 0.0)
    nu = _dot(t, w2[...]) + b2[...]
    hn = _ln(h[...] + nu, g[...], bn[...])
    h_o[...] = hn
    as_o[...] = _dot(hn, ws[...])
    ar_o[...] = _dot(hn, wr[...])


def _node_up_dec_body(h, a0, a1, wh, wa, b1, w2, b2, g, bn, wd1, c1, wd2, c2,
                      out_o):
    agg = a0[...] + a1[...]
    t = jnp.maximum(_dot(h[...], wh[...]) + _dot(agg, wa[...]) + b1[...], 0.0)
    nu = _dot(t, w2[...]) + b2[...]
    hn = _ln(h[...] + nu, g[...], bn[...])
    d = jnp.maximum(_dot(hn, wd1[...]) + c1[...], 0.0)
    out_o[...] = _dot(d, wd2[...]) + c2[...]


def _full(a):
    nd = len(a.shape)
    return pl.BlockSpec(a.shape, lambda i: (0,) * nd)


def _rows(tile, width):
    return pl.BlockSpec((tile, width), lambda i: (i, 0))


def _tc_call(body, row_args, consts, n_rows, tile, out_widths):
    grid = (n_rows // tile,)
    in_specs = [_rows(tile, a.shape[-1]) for a in row_args]
    in_specs += [_full(c) for c in consts]
    out_shape = [jax.ShapeDtypeStruct((n_rows, w), jnp.float32)
                 for w in out_widths]
    out_specs = [_rows(tile, w) for w in out_widths]
    return pl.pallas_call(
        body, grid=grid, in_specs=in_specs, out_specs=out_specs,
        out_shape=out_shape)(*row_args, *consts)


def _sc_gather(as_t, ar_t, ve, s3, r3, off0):
    eh = s3.shape[0] * s3.shape[1] * s3.shape[2]
    per_w = eh // NW
    n_ch = per_w // CHUNK
    mesh = plsc.VectorSubcoreMesh(core_axis_name="c", subcore_axis_name="s")

    @functools.partial(
        pl.kernel, mesh=mesh,
        out_type=jax.ShapeDtypeStruct((eh, D), jnp.float32),
        scratch_types=[
            pltpu.VMEM((n_ch, CHUNK), jnp.int32),
            pltpu.VMEM((n_ch, CHUNK), jnp.int32),
            pltpu.VMEM((2, CHUNK, D), jnp.float32),
            pltpu.VMEM((2, CHUNK, D), jnp.float32),
            pltpu.VMEM((2, CHUNK, D), jnp.float32),
            pltpu.SemaphoreType.DMA,
            pltpu.SemaphoreType.DMA,
            pltpu.SemaphoreType.DMA,
            pltpu.SemaphoreType.DMA,
        ],
    )
    def k(as_h, ar_h, ve_h, si_h, ri_h, out_h, si_v, ri_v, rs_v, rr_v, ve_v,
          g0, g1, o0, o1):
        wid = lax.axis_index("s") * NC + lax.axis_index("c")
        base = wid * per_w
        gsem = (g0, g1)
        osem = (o0, o1)

        pltpu.sync_copy(si_h.at[wid], si_v)
        pltpu.sync_copy(ri_h.at[wid], ri_v)

        def issue_g(b, c):
            pltpu.async_copy(as_h.at[si_v.at[c]], rs_v.at[b], gsem[b])
            pltpu.async_copy(ar_h.at[ri_v.at[c]], rr_v.at[b], gsem[b])
            pltpu.async_copy(ve_h.at[pl.ds(off0 + base + c * CHUNK, CHUNK)],
                             ve_v.at[b], gsem[b])

        def wait_g(b):
            for dst in (rs_v, rr_v, ve_v):
                pltpu.make_async_copy(ve_h.at[pl.ds(base, CHUNK)],
                                      dst.at[b], gsem[b]).wait()

        def wait_o(b):
            pltpu.make_async_copy(ve_v.at[b],
                                  out_h.at[pl.ds(base, CHUNK)],
                                  osem[b]).wait()

        def compute(b):
            @plsc.parallel_loop(0, CHUNK, unroll=4)
            def _row(j):
                for kk in range(D // 16):
                    sl = pl.ds(kk * 16, 16)
                    v = (rs_v[b, j, sl] + rr_v[b, j, sl]
                         + ve_v[b, j, sl])
                    ve_v[b, j, sl] = jnp.maximum(v, 0.0)

        issue_g(0, 0)

        def pair(i, carry):
            for b in (0, 1):
                c = 2 * i + b
                other = 1 - b

                @pl.when(c + 1 < n_ch)
                def _prefetch():
                    @pl.when(c >= 1)
                    def _drain():
                        wait_o(other)

                    issue_g(other, c + 1)

                wait_g(b)
                compute(b)
                pltpu.async_copy(ve_v.at[b],
                                 out_h.at[pl.ds(base + c * CHUNK, CHUNK)],
                                 osem[b])
            return carry

        lax.fori_loop(0, n_ch // 2, pair, 0)
        if n_ch % 2 == 1:
            wait_g(0)
            compute(0)
            pltpu.async_copy(
                ve_v.at[0],
                out_h.at[pl.ds(base + (n_ch - 1) * CHUNK, CHUNK)], osem[0])
        wait_o(0)
        wait_o(1)

    return k(as_t, ar_t, ve, s3, r3)


def _sc_scatter(m, r3, init0, init1, n_pad):
    e = m.shape[0]
    per_w = e // NW
    n_ch = per_w // CHUNK
    rows_per_tile = n_pad // NS
    mesh = plsc.VectorSubcoreMesh(core_axis_name="c", subcore_axis_name="s")

    @functools.partial(
        pl.kernel, mesh=mesh,
        out_type=[jax.ShapeDtypeStruct((n_pad, D), jnp.float32),
                  jax.ShapeDtypeStruct((n_pad, D), jnp.float32)],
        scratch_types=[
            pltpu.VMEM_SHARED((n_pad, D), jnp.float32),
            pltpu.VMEM((n_ch, CHUNK), jnp.int32),
            pltpu.VMEM((2, CHUNK, D), jnp.float32),
            pltpu.SemaphoreType.DMA,
            pltpu.SemaphoreType.DMA,
            pltpu.SemaphoreType.DMA,
            pltpu.SemaphoreType.DMA,
        ],
    )
    def k(m_h, ri_h, z0_h, z1_h, out0_h, out1_h, acc_sh, ri_v, m_v,
          l0, l1, s0, s1):
        cid = lax.axis_index("c")
        sid = lax.axis_index("s")
        wid = sid * NC + cid
        base = wid * per_w
        lsem = (l0, l1)
        ssem = (s0, s1)

        @pl.when((sid == 0) & (cid == 0))
        def _init0():
            pltpu.sync_copy(z0_h, acc_sh)

        @pl.when((sid == 0) & (cid == 1))
        def _init1():
            pltpu.sync_copy(z1_h, acc_sh)

        pltpu.sync_copy(ri_h.at[wid], ri_v)
        plsc.subcore_barrier()

        def issue_m(b, c):
            pltpu.async_copy(m_h.at[pl.ds(base + c * CHUNK, CHUNK)],
                             m_v.at[b], lsem[b])

        def wait_m(b):
            pltpu.make_async_copy(m_h.at[pl.ds(base, CHUNK)],
                                  m_v.at[b], lsem[b]).wait()

        def wait_sc(b):
            pltpu.make_async_copy(m_v.at[b], acc_sh.at[pl.ds(0, CHUNK)],
                                  ssem[b]).wait()

        issue_m(0, 0)

        def pair(i, carry):
            for b in (0, 1):
                c = 2 * i + b
                other = 1 - b

                @pl.when(c + 1 < n_ch)
                def _prefetch():
                    @pl.when(c >= 1)
                    def _drain():
                        wait_sc(other)

                    issue_m(other, c + 1)

                wait_m(b)
                pltpu.async_copy(m_v.at[b], acc_sh.at[ri_v.at[c]], ssem[b],
                                 add=True)
            return carry

        lax.fori_loop(0, n_ch // 2, pair, 0)
        if n_ch % 2 == 1:
            wait_m(0)
            pltpu.async_copy(m_v.at[0], acc_sh.at[ri_v.at[n_ch - 1]],
                             ssem[0], add=True)
        wait_sc(0)
        wait_sc(1)
        plsc.subcore_barrier()

        row0 = sid * rows_per_tile

        @pl.when(cid == 0)
        def _wb0():
            pltpu.sync_copy(acc_sh.at[pl.ds(row0, rows_per_tile)],
                            out0_h.at[pl.ds(row0, rows_per_tile)])

        @pl.when(cid == 1)
        def _wb1():
            pltpu.sync_copy(acc_sh.at[pl.ds(row0, rows_per_tile)],
                            out1_h.at[pl.ds(row0, rows_per_tile)])

    return k(m, r3, init0, init1)


def kernel(x, edge_index, edge_attr, params):
    n = x.shape[0]
    e = edge_index.shape[1]
    n_pad = ((n + 8 * NS - 1) // (8 * NS)) * (8 * NS)

    eh = e // 2
    per_w = eh // NW
    n_ch = per_w // CHUNK
    s_idx = edge_index[0].astype(jnp.int32)
    r_idx = edge_index[1].astype(jnp.int32)
    s3 = [s_idx[:eh].reshape(NW, n_ch, CHUNK),
          s_idx[eh:].reshape(NW, n_ch, CHUNK)]
    r3 = [r_idx[:eh].reshape(NW, n_ch, CHUNK),
          r_idx[eh:].reshape(NW, n_ch, CHUNK)]

    def lin(p):
        return p["W"], p["b"].reshape(1, -1)

    ne1w, ne1b = lin(params["node_enc"][0])
    ne2w, ne2b = lin(params["node_enc"][1])
    ng = params["node_enc_ln"]["g"].reshape(1, -1)
    nb = params["node_enc_ln"]["b"].reshape(1, -1)
    ee1w, ee1b = lin(params["edge_enc"][0])
    ee2w, ee2b = lin(params["edge_enc"][1])
    eg = params["edge_enc_ln"]["g"].reshape(1, -1)
    eb = params["edge_enc_ln"]["b"].reshape(1, -1)

    lw = []
    for lp in params["layers"]:
        em1w, em1b = lin(lp["edge_mlp"][0])
        em2w, em2b = lin(lp["edge_mlp"][1])
        nm1w, nm1b = lin(lp["node_mlp"][0])
        nm2w, nm2b = lin(lp["node_mlp"][1])
        lw.append(dict(
            ws=em1w[:D], wr=em1w[D:2 * D], we=em1w[2 * D:], c1=em1b,
            em2w=em2w, em2b=em2b,
            eg=lp["edge_norm"]["g"].reshape(1, -1),
            eb=lp["edge_norm"]["b"].reshape(1, -1),
            wh=nm1w[:D], wa=nm1w[D:], nb1=nm1b,
            nm2w=nm2w, nm2b=nm2b,
            ng=lp["node_norm"]["g"].reshape(1, -1),
            nb2=lp["node_norm"]["b"].reshape(1, -1),
        ))

    d1w, d1b = lin(params["dec"][0])
    d2w, d2b = lin(params["dec"][1])
    d2w = jnp.pad(d2w, ((0, 0), (0, D - d2w.shape[1])))
    d2b = jnp.pad(d2b, ((0, 0), (0, D - d2b.shape[1])))

    tn, te = 1000, 2000

    h, as_t, ar_t = _tc_call(
        _node_enc_body, [x],
        [ne1w, ne1b, ne2w, ne2b, ng, nb, lw[0]["ws"], lw[0]["wr"]],
        n, tn, [D, D, D])

    ve0, ve1 = _tc_call(
        _edge_proj_body, [edge_attr],
        [ee1w, ee1b, ee2w, ee2b, eg, eb,
         lw[0]["we"], lw[0]["c1"], lw[1]["we"], lw[1]["c1"]],
        e, te, [D, D])

    zeros = jnp.zeros((n_pad, D), jnp.float32)
    ve = [ve0, ve1]
    out = None
    for l, w in enumerate(lw):
        m1a = _sc_gather(as_t, ar_t, ve[l], s3[0], r3[0], 0)
        m1b = _sc_gather(as_t, ar_t, ve[l], s3[1], r3[1], eh)
        ma = _tc_call(_edge_m_body, [m1a],
                      [w["em2w"], w["em2b"], w["eg"], w["eb"]],
                      eh, te, [D])[0]
        mb = _tc_call(_edge_m_body, [m1b],
                      [w["em2w"], w["em2b"], w["eg"], w["eb"]],
                      eh, te, [D])[0]
        pa0, pa1 = _sc_scatter(ma, r3[0], zeros, zeros, n_pad)
        a0, a1 = _sc_scatter(mb, r3[1], pa0, pa1, n_pad)
        if l == 0:
            nxt = lw[1]
            h, as_t, ar_t = _tc_call(
                _node_up_body, [h, a0[:n], a1[:n]],
                [w["wh"], w["wa"], w["nb1"], w["nm2w"], w["nm2b"],
                 w["ng"], w["nb2"], nxt["ws"], nxt["wr"]],
                n, tn, [D, D, D])
        else:
            out = _tc_call(
                _node_up_dec_body, [h, a0[:n], a1[:n]],
                [w["wh"], w["wa"], w["nb1"], w["nm2w"], w["nm2b"],
                 w["ng"], w["nb2"], d1w, d1b, d2w, d2b],
                n, tn, [D])[0]

    return out[:, :3]


# speedup vs baseline: 1.2418x; 1.0524x over previous
import functools

import jax
import jax.numpy as jnp
from jax import lax
from jax.experimental import pallas as pl
from jax.experimental.pallas import tpu as pltpu
from jax.experimental.pallas import tpu_sc as plsc

D = 128
EPS = 1e-5

NC = 2
NS = 16
NW = NC * NS
CHUNK = 40


def _ln(y, g, b):
    mu = jnp.mean(y, axis=-1, keepdims=True)
    d = y - mu
    var = jnp.mean(d * d, axis=-1, keepdims=True)
    return d * lax.rsqrt(var + EPS) * g + b


def _dot(a, b):
    return jnp.dot(a, b, preferred_element_type=jnp.float32)


def _node_enc_body(x, w1, b1, w2, b2, g, bn, ws, wr, h_o, as_o, ar_o):
    t = jnp.maximum(_dot(x[...], w1[...]) + b1[...], 0.0)
    h = _ln(_dot(t, w2[...]) + b2[...], g[...], bn[...])
    h_o[...] = h
    as_o[...] = _dot(h, ws[...])
    ar_o[...] = _dot(h, wr[...])


def _edge_proj_body(ea, w1, b1, w2, b2, g, bn, we0, c0, we1, c1, ve0_o, ve1_o):
    t = jnp.maximum(_dot(ea[...], w1[...]) + b1[...], 0.0)
    e = _ln(_dot(t, w2[...]) + b2[...], g[...], bn[...])
    ve0_o[...] = _dot(e, we0[...]) + c0[...]
    ve1_o[...] = _dot(e, we1[...]) + c1[...]


def _edge_m_body(m1, vex, w2, b2, g, bn, m_o):
    t = jnp.maximum(m1[...] + vex[...], 0.0)
    m_o[...] = _ln(jnp.maximum(_dot(t, w2[...]) + b2[...], 0.0),
                   g[...], bn[...])


def _node_up_body(h, a0, a1, wh, wa, b1, w2, b2, g, bn, ws, wr,
                  h_o, as_o, ar_o):
    agg = a0[...] + a1[...]
    t = jnp.maximum(_dot(h[...], wh[...]) + _dot(agg, wa[...]) + b1[...], 0.0)
    nu = _dot(t, w2[...]) + b2[...]
    hn = _ln(h[...] + nu, g[...], bn[...])
    h_o[...] = hn
    as_o[...] = _dot(hn, ws[...])
    ar_o[...] = _dot(hn, wr[...])


def _node_up_dec_body(h, a0, a1, wh, wa, b1, w2, b2, g, bn, wd1, c1, wd2, c2,
                      out_o):
    agg = a0[...] + a1[...]
    t = jnp.maximum(_dot(h[...], wh[...]) + _dot(agg, wa[...]) + b1[...], 0.0)
    nu = _dot(t, w2[...]) + b2[...]
    hn = _ln(h[...] + nu, g[...], bn[...])
    d = jnp.maximum(_dot(hn, wd1[...]) + c1[...], 0.0)
    out_o[...] = _dot(d, wd2[...]) + c2[...]


def _full(a):
    nd = len(a.shape)
    return pl.BlockSpec(a.shape, lambda i: (0,) * nd)


def _rows(tile, width):
    return pl.BlockSpec((tile, width), lambda i: (i, 0))


def _tc_call(body, row_args, consts, n_rows, tile, out_widths):
    grid = (n_rows // tile,)
    in_specs = [_rows(tile, a.shape[-1]) for a in row_args]
    in_specs += [_full(c) for c in consts]
    out_shape = [jax.ShapeDtypeStruct((n_rows, w), jnp.float32)
                 for w in out_widths]
    out_specs = [_rows(tile, w) for w in out_widths]
    return pl.pallas_call(
        body, grid=grid, in_specs=in_specs, out_specs=out_specs,
        out_shape=out_shape)(*row_args, *consts)


def _sc_gather(as_t, ar_t, s3, r3):
    eh = s3.shape[0] * s3.shape[1] * s3.shape[2]
    per_w = eh // NW
    n_ch = per_w // CHUNK
    mesh = plsc.VectorSubcoreMesh(core_axis_name="c", subcore_axis_name="s")

    @functools.partial(
        pl.kernel, mesh=mesh,
        out_type=jax.ShapeDtypeStruct((eh, D), jnp.float32),
        scratch_types=[
            pltpu.VMEM((n_ch, CHUNK), jnp.int32),
            pltpu.VMEM((n_ch, CHUNK), jnp.int32),
            pltpu.VMEM((2, CHUNK, D), jnp.float32),
            pltpu.VMEM((2, CHUNK, D), jnp.float32),
            pltpu.SemaphoreType.DMA,
            pltpu.SemaphoreType.DMA,
            pltpu.SemaphoreType.DMA,
            pltpu.SemaphoreType.DMA,
        ],
    )
    def k(as_h, ar_h, si_h, ri_h, out_h, si_v, ri_v, rs_v, rr_v,
          g0, g1, o0, o1):
        wid = lax.axis_index("s") * NC + lax.axis_index("c")
        base = wid * per_w
        gsem = (g0, g1)
        osem = (o0, o1)

        pltpu.sync_copy(si_h.at[wid], si_v)
        pltpu.sync_copy(ri_h.at[wid], ri_v)

        def issue_g(b, c):
            pltpu.async_copy(as_h.at[si_v.at[c]], rs_v.at[b], gsem[b])
            pltpu.async_copy(ar_h.at[ri_v.at[c]], rr_v.at[b], gsem[b])

        def wait_g(b):
            for dst in (rs_v, rr_v):
                pltpu.make_async_copy(as_h.at[pl.ds(0, CHUNK)],
                                      dst.at[b], gsem[b]).wait()

        def wait_o(b):
            pltpu.make_async_copy(rs_v.at[b],
                                  out_h.at[pl.ds(base, CHUNK)],
                                  osem[b]).wait()

        def compute(b):
            @plsc.parallel_loop(0, CHUNK, unroll=4)
            def _row(j):
                for kk in range(D // 16):
                    sl = pl.ds(kk * 16, 16)
                    rs_v[b, j, sl] = rs_v[b, j, sl] + rr_v[b, j, sl]

        issue_g(0, 0)

        def pair(i, carry):
            for b in (0, 1):
                c = 2 * i + b
                other = 1 - b

                @pl.when(c + 1 < n_ch)
                def _prefetch():
                    @pl.when(c >= 1)
                    def _drain():
                        wait_o(other)

                    issue_g(other, c + 1)

                wait_g(b)
                compute(b)
                pltpu.async_copy(rs_v.at[b],
                                 out_h.at[pl.ds(base + c * CHUNK, CHUNK)],
                                 osem[b])
            return carry

        lax.fori_loop(0, n_ch // 2, pair, 0)
        if n_ch % 2 == 1:
            wait_g(0)
            compute(0)
            pltpu.async_copy(
                rs_v.at[0],
                out_h.at[pl.ds(base + (n_ch - 1) * CHUNK, CHUNK)], osem[0])
        wait_o(0)
        wait_o(1)

    return k(as_t, ar_t, s3, r3)


def _sc_scatter(m, r3, init0, init1, n_pad):
    e = m.shape[0]
    per_w = e // NW
    n_ch = per_w // CHUNK
    rows_per_tile = n_pad // NS
    mesh = plsc.VectorSubcoreMesh(core_axis_name="c", subcore_axis_name="s")

    @functools.partial(
        pl.kernel, mesh=mesh,
        out_type=[jax.ShapeDtypeStruct((n_pad, D), jnp.float32),
                  jax.ShapeDtypeStruct((n_pad, D), jnp.float32)],
        scratch_types=[
            pltpu.VMEM_SHARED((n_pad, D), jnp.float32),
            pltpu.VMEM((n_ch, CHUNK), jnp.int32),
            pltpu.VMEM((2, CHUNK, D), jnp.float32),
            pltpu.SemaphoreType.DMA,
            pltpu.SemaphoreType.DMA,
            pltpu.SemaphoreType.DMA,
            pltpu.SemaphoreType.DMA,
        ],
    )
    def k(m_h, ri_h, z0_h, z1_h, out0_h, out1_h, acc_sh, ri_v, m_v,
          l0, l1, s0, s1):
        cid = lax.axis_index("c")
        sid = lax.axis_index("s")
        wid = sid * NC + cid
        base = wid * per_w
        lsem = (l0, l1)
        ssem = (s0, s1)

        @pl.when((sid == 0) & (cid == 0))
        def _init0():
            pltpu.sync_copy(z0_h, acc_sh)

        @pl.when((sid == 0) & (cid == 1))
        def _init1():
            pltpu.sync_copy(z1_h, acc_sh)

        pltpu.sync_copy(ri_h.at[wid], ri_v)
        plsc.subcore_barrier()

        def issue_m(b, c):
            pltpu.async_copy(m_h.at[pl.ds(base + c * CHUNK, CHUNK)],
                             m_v.at[b], lsem[b])

        def wait_m(b):
            pltpu.make_async_copy(m_h.at[pl.ds(base, CHUNK)],
                                  m_v.at[b], lsem[b]).wait()

        def wait_sc(b):
            pltpu.make_async_copy(m_v.at[b], acc_sh.at[pl.ds(0, CHUNK)],
                                  ssem[b]).wait()

        issue_m(0, 0)

        def pair(i, carry):
            for b in (0, 1):
                c = 2 * i + b
                other = 1 - b

                @pl.when(c + 1 < n_ch)
                def _prefetch():
                    @pl.when(c >= 1)
                    def _drain():
                        wait_sc(other)

                    issue_m(other, c + 1)

                wait_m(b)
                pltpu.async_copy(m_v.at[b], acc_sh.at[ri_v.at[c]], ssem[b],
                                 add=True)
            return carry

        lax.fori_loop(0, n_ch // 2, pair, 0)
        if n_ch % 2 == 1:
            wait_m(0)
            pltpu.async_copy(m_v.at[0], acc_sh.at[ri_v.at[n_ch - 1]],
                             ssem[0], add=True)
        wait_sc(0)
        wait_sc(1)
        plsc.subcore_barrier()

        row0 = sid * rows_per_tile

        @pl.when(cid == 0)
        def _wb0():
            pltpu.sync_copy(acc_sh.at[pl.ds(row0, rows_per_tile)],
                            out0_h.at[pl.ds(row0, rows_per_tile)])

        @pl.when(cid == 1)
        def _wb1():
            pltpu.sync_copy(acc_sh.at[pl.ds(row0, rows_per_tile)],
                            out1_h.at[pl.ds(row0, rows_per_tile)])

    return k(m, r3, init0, init1)


def kernel(x, edge_index, edge_attr, params):
    n = x.shape[0]
    e = edge_index.shape[1]
    n_pad = ((n + 8 * NS - 1) // (8 * NS)) * (8 * NS)

    eh = e // 2
    per_w = eh // NW
    n_ch = per_w // CHUNK
    s_idx = edge_index[0].astype(jnp.int32)
    r_idx = edge_index[1].astype(jnp.int32)
    s3 = [s_idx[:eh].reshape(NW, n_ch, CHUNK),
          s_idx[eh:].reshape(NW, n_ch, CHUNK)]
    r3 = [r_idx[:eh].reshape(NW, n_ch, CHUNK),
          r_idx[eh:].reshape(NW, n_ch, CHUNK)]

    def lin(p):
        return p["W"], p["b"].reshape(1, -1)

    ne1w, ne1b = lin(params["node_enc"][0])
    ne2w, ne2b = lin(params["node_enc"][1])
    ng = params["node_enc_ln"]["g"].reshape(1, -1)
    nb = params["node_enc_ln"]["b"].reshape(1, -1)
    ee1w, ee1b = lin(params["edge_enc"][0])
    ee2w, ee2b = lin(params["edge_enc"][1])
    eg = params["edge_enc_ln"]["g"].reshape(1, -1)
    eb = params["edge_enc_ln"]["b"].reshape(1, -1)

    lw = []
    for lp in params["layers"]:
        em1w, em1b = lin(lp["edge_mlp"][0])
        em2w, em2b = lin(lp["edge_mlp"][1])
        nm1w, nm1b = lin(lp["node_mlp"][0])
        nm2w, nm2b = lin(lp["node_mlp"][1])
        lw.append(dict(
            ws=em1w[:D], wr=em1w[D:2 * D], we=em1w[2 * D:], c1=em1b,
            em2w=em2w, em2b=em2b,
            eg=lp["edge_norm"]["g"].reshape(1, -1),
            eb=lp["edge_norm"]["b"].reshape(1, -1),
            wh=nm1w[:D], wa=nm1w[D:], nb1=nm1b,
            nm2w=nm2w, nm2b=nm2b,
            ng=lp["node_norm"]["g"].reshape(1, -1),
            nb2=lp["node_norm"]["b"].reshape(1, -1),
        ))

    d1w, d1b = lin(params["dec"][0])
    d2w, d2b = lin(params["dec"][1])
    d2w = jnp.pad(d2w, ((0, 0), (0, D - d2w.shape[1])))
    d2b = jnp.pad(d2b, ((0, 0), (0, D - d2b.shape[1])))

    tn, te = 1000, 2000

    h, as_t, ar_t = _tc_call(
        _node_enc_body, [x],
        [ne1w, ne1b, ne2w, ne2b, ng, nb, lw[0]["ws"], lw[0]["wr"]],
        n, tn, [D, D, D])

    ep_consts = [ee1w, ee1b, ee2w, ee2b, eg, eb,
                 lw[0]["we"], lw[0]["c1"], lw[1]["we"], lw[1]["c1"]]
    ve0a, ve1a = _tc_call(_edge_proj_body, [edge_attr[:eh]], ep_consts,
                          eh, te, [D, D])
    ve0b, ve1b = _tc_call(_edge_proj_body, [edge_attr[eh:]], ep_consts,
                          eh, te, [D, D])

    zeros = jnp.zeros((n_pad, D), jnp.float32)
    ve = [(ve0a, ve0b), (ve1a, ve1b)]
    out = None
    for l, w in enumerate(lw):
        m1a = _sc_gather(as_t, ar_t, s3[0], r3[0])
        m1b = _sc_gather(as_t, ar_t, s3[1], r3[1])
        ma = _tc_call(_edge_m_body, [m1a, ve[l][0]],
                      [w["em2w"], w["em2b"], w["eg"], w["eb"]],
                      eh, te, [D])[0]
        mb = _tc_call(_edge_m_body, [m1b, ve[l][1]],
                      [w["em2w"], w["em2b"], w["eg"], w["eb"]],
                      eh, te, [D])[0]
        pa0, pa1 = _sc_scatter(ma, r3[0], zeros, zeros, n_pad)
        a0, a1 = _sc_scatter(mb, r3[1], pa0, pa1, n_pad)
        if l == 0:
            nxt = lw[1]
            h, as_t, ar_t = _tc_call(
                _node_up_body, [h, a0[:n], a1[:n]],
                [w["wh"], w["wa"], w["nb1"], w["nm2w"], w["nm2b"],
                 w["ng"], w["nb2"], nxt["ws"], nxt["wr"]],
                n, tn, [D, D, D])
        else:
            out = _tc_call(
                _node_up_dec_body, [h, a0[:n], a1[:n]],
                [w["wh"], w["wa"], w["nb1"], w["nm2w"], w["nm2b"],
                 w["ng"], w["nb2"], d1w, d1b, d2w, d2b],
                n, tn, [D])[0]

    return out[:, :3]
